# baseline (speedup 1.0000x reference)
import functools

import jax
import jax.numpy as jnp
import numpy as np
from jax import lax
from jax.experimental import pallas as pl
from jax.experimental.pallas import tpu as pltpu
from jax.experimental.pallas import tpu_sc as plsc

F = 26
V = 3846
S = 99996
D = 16
B = 4096
NT = 32
RPT = B // NT
G = 32
NP = 4
FC = 26
NBUF = 4


def _sc_ffm(t0, t1, t2, t3, xoffT):
    mesh = plsc.VectorSubcoreMesh(core_axis_name="c", subcore_axis_name="s")

    @functools.partial(
        pl.kernel,
        out_type=jax.ShapeDtypeStruct((B * D,), jnp.float32),
        mesh=mesh,
        scratch_types=[
            pltpu.VMEM((G, RPT), jnp.int32),
            pltpu.VMEM((NBUF * G,), jnp.int32),
            pltpu.VMEM((NBUF, NP, G, 128), jnp.float32),
            pltpu.VMEM((RPT * D,), jnp.float32),
            pltpu.SemaphoreType.DMA,
            pltpu.SemaphoreType.DMA,
            pltpu.SemaphoreType.DMA,
            pltpu.SemaphoreType.DMA,
        ],
        compiler_params=pltpu.CompilerParams(
            use_tc_tiling_on_sc=False, needs_layout_passes=False),
    )
    def kern(t0_hbm, t1_hbm, t2_hbm, t3_hbm, xo_hbm, z_hbm,
             xoff_v, idx_v, gbuf, zloc, sem0, sem1, sem2, sem3):
        wid = lax.axis_index("s") * 2 + lax.axis_index("c")
        base = wid * RPT
        pltpu.sync_copy(xo_hbm.at[:, pl.ds(base, RPT)], xoff_v)
        tabs = (t0_hbm, t1_hbm, t2_hbm, t3_hbm)
        sems = (sem0, sem1, sem2, sem3)

        lanes = lax.iota(jnp.int32, 16)

        def fire(b, r):
            rv = jnp.full((16,), r, jnp.int32)
            xv0 = plsc.load_gather(xoff_v, [lanes, rv])
            xv1 = plsc.load_gather(xoff_v, [lanes + D, rv])
            idx_v[pl.ds(b * G, D)] = xv0
            idx_v[pl.ds(b * G + D, D)] = jnp.minimum(xv1, S - 1)
            for p in range(NP):
                for h in range(2):
                    pltpu.async_copy(
                        tabs[p].at[idx_v.at[pl.ds(b * G + h * D, D)]],
                        gbuf.at[b, p].at[pl.ds(h * D, D)], sems[b])

        def wait(b):
            for p in range(NP):
                for h in range(2):
                    pltpu.make_async_copy(
                        tabs[p].at[idx_v.at[pl.ds(b * G + h * D, D)]],
                        gbuf.at[b, p].at[pl.ds(h * D, D)], sems[b]).wait()

        for b in range(NBUF):
            fire(b, b)

        @pl.loop(0, RPT, step=NBUF)
        def _(r):
            for b in range(NBUF):
                wait(b)
                acc = gbuf[b, FC // 8, 0, pl.ds((FC % 8) * D, D)]
                for f in range(1, F):
                    acc = acc + gbuf[b, FC // 8, f, pl.ds((FC % 8) * D, D)]
                for i in range(F - 1):
                    for j in range(i + 1, F):
                        va = gbuf[b, j // 8, i, pl.ds((j % 8) * D, D)]
                        vb = gbuf[b, i // 8, j, pl.ds((i % 8) * D, D)]
                        acc = acc + va * vb
                zloc[pl.ds((r + b) * D, D)] = acc

                @pl.when(r + b + NBUF < RPT)
                def _():
                    fire(b, r + b + NBUF)

        pltpu.sync_copy(zloc, z_hbm.at[pl.ds(base * D, RPT * D)])

    return kern(t0, t1, t2, t3, xoffT)


def _tc_finish(z2d, bias):
    def body(z_ref, b_ref, o_ref):
        o_ref[...] = jax.nn.sigmoid(jnp.sum(z_ref[...], axis=1) + b_ref[0])

    return pl.pallas_call(
        body,
        out_shape=jax.ShapeDtypeStruct((B,), jnp.float32),
    )(z2d, bias)


@jax.jit
def kernel(x, fc_weight, bias, ffm_tables):
    offsets = np.arange(F, dtype=np.int32) * V
    x_off = x.astype(jnp.int32) + jnp.asarray(offsets)[None, :]
    xoffT = jnp.concatenate(
        [x_off.T, jnp.full((G - F, B), S, jnp.int32)], axis=0)

    packs = []
    for i in range(3):
        packs.append(
            ffm_tables[8 * i:8 * i + 8].transpose(1, 0, 2).reshape(S, 128))
    fc16 = jnp.concatenate([fc_weight, jnp.zeros((S, D - 1), jnp.float32)], 1)
    last = jnp.concatenate([ffm_tables[24:26], fc16[None]], axis=0)
    t3 = jnp.pad(last.transpose(1, 0, 2).reshape(S, 48), ((0, 0), (0, 80)))
    packs.append(t3)

    z = _sc_ffm(*packs, xoffT)
    return _tc_finish(z.reshape(B, D), bias)
